# baseline (speedup 1.0000x reference)
import functools

import jax
import jax.numpy as jnp
from jax.experimental import pallas as pl
from jax.experimental.pallas import tpu as pltpu

_INTERPRET = False


def _encoder_kernel(xh_ref, w1h_ref, b1_ref, hh_ref):
    acc = jnp.dot(xh_ref[...], w1h_ref[...],
                  preferred_element_type=jnp.float32)
    hh_ref[...] = jnp.tanh(acc + b1_ref[...]).astype(jnp.bfloat16)


def _main_kernel(hh_ref, w2h_ref, b2_ref, g_ref, cb_ref, s_ref, res_ref,
                 zbuf_ref, *, nm, K, nj):
    j = pl.program_id(1)
    par = jax.lax.rem(j, 2)

    zprev = zbuf_ref[1 - par]
    a = jnp.maximum(zprev + b2_ref[...], 0.0)
    zg = a + g_ref[...]
    for mi in range(nm):
        zm = zg[:, mi * K:(mi + 1) * K]
        mx = jnp.max(zm, axis=-1, keepdims=True)
        iota = jax.lax.broadcasted_iota(jnp.int32, zm.shape, 1).astype(
            jnp.float32)
        idx = jnp.min(jnp.where(zm == mx, iota, float(K)), axis=-1,
                      keepdims=True)
        oh = (iota == idx).astype(jnp.float32)
        s_ref[:, mi * K:(mi + 1) * K] = oh

    part = jnp.dot(hh_ref[...], w2h_ref[...],
                   preferred_element_type=jnp.float32)
    zbuf_ref[par] = part

    for mi in range(nm):
        ohb = s_ref[:, mi * K:(mi + 1) * K].astype(jnp.bfloat16)
        r = jnp.dot(ohb, cb_ref[mi], preferred_element_type=jnp.float32)
        sub = r.shape[-1] // 2
        res_ref[0, :, mi * sub:(mi + 1) * sub] = r[:, :sub] + r[:, sub:]


def kernel(x, W1, b1, W2, b2, codebook, gumbel):
    B, DIM = x.shape
    MK2 = W1.shape[1]
    MK = W2.shape[1]
    M, K, SUB = codebook.shape

    xh = x.astype(jnp.bfloat16)
    w1h = W1.astype(jnp.bfloat16)
    w2h = W2.astype(jnp.bfloat16)
    cbh = codebook.astype(jnp.bfloat16)
    cbl = (codebook - cbh.astype(jnp.float32)).astype(jnp.bfloat16)
    cb_cat = jnp.concatenate([cbh, cbl], axis=-1)

    B1 = 1024 if B % 1024 == 0 else B
    grid_a = (B // B1,)
    hh = pl.pallas_call(
        _encoder_kernel,
        grid=grid_a,
        in_specs=[
            pl.BlockSpec((B1, DIM), lambda i: (i, 0)),
            pl.BlockSpec((DIM, MK2), lambda i: (0, 0)),
            pl.BlockSpec((1, MK2), lambda i: (0, 0)),
        ],
        out_specs=pl.BlockSpec((B1, MK2), lambda i: (i, 0)),
        out_shape=jax.ShapeDtypeStruct((B, MK2), jnp.bfloat16),
        compiler_params=pltpu.CompilerParams(
            dimension_semantics=("parallel",),
            vmem_limit_bytes=50 * 1024 * 1024,
        ),
        name="wolfpq_encoder",
        interpret=_INTERPRET,
    )(xh, w1h, b1.reshape(1, MK2))

    BB = 1024 if B % 1024 == 0 else B
    NB = 512 if MK % 512 == 0 else MK
    nm = NB // K
    nj = MK // NB
    nhalves = 2 if (B // BB) % 2 == 0 else 1
    Bh = B // nhalves

    def prev_i(i, j):
        return jnp.where(j == 0, jnp.maximum(i - 1, 0), i)

    def prev_j(j):
        return jax.lax.rem(j + nj - 1, nj)

    def cur_j(j):
        return jnp.minimum(j, nj - 1)

    body = functools.partial(_main_kernel, nm=nm, K=K, nj=nj)

    def run_b(hh_half, g2d_half):
        return pl.pallas_call(
            body,
            grid=(Bh // BB, nj + 1),
            in_specs=[
                pl.BlockSpec((BB, MK2), lambda i, j: (i, 0)),
                pl.BlockSpec((MK2, NB), lambda i, j: (0, cur_j(j))),
                pl.BlockSpec((1, NB), lambda i, j: (0, prev_j(j))),
                pl.BlockSpec((BB, NB),
                             lambda i, j: (prev_i(i, j), prev_j(j))),
                pl.BlockSpec((nm, K, 2 * SUB),
                             lambda i, j: (prev_j(j), 0, 0)),
            ],
            out_specs=[
                pl.BlockSpec((BB, NB),
                             lambda i, j: (prev_i(i, j), prev_j(j))),
                pl.BlockSpec((1, BB, nm * SUB),
                             lambda i, j: (prev_j(j), prev_i(i, j), 0)),
            ],
            out_shape=[
                jax.ShapeDtypeStruct((Bh, MK), jnp.float32),
                jax.ShapeDtypeStruct((nj, Bh, nm * SUB), jnp.float32),
            ],
            scratch_shapes=[pltpu.VMEM((2, BB, NB), jnp.float32)],
            compiler_params=pltpu.CompilerParams(
                dimension_semantics=("parallel", "arbitrary"),
                vmem_limit_bytes=56 * 1024 * 1024,
            ),
            name="wolfpq_vq",
            interpret=_INTERPRET,
        )(hh_half, w2h, b2.reshape(1, MK), g2d_half, cb_cat)

    s_parts, res_parts = [], []
    for half in range(nhalves):
        rows = slice(half * Bh, (half + 1) * Bh)
        s2d_h, res_h = run_b(hh[rows], gumbel[rows].reshape(Bh, MK))
        s_parts.append(s2d_h.reshape(Bh, M, K))
        res_parts.append(res_h.transpose(1, 0, 2).reshape(Bh, M * SUB))

    res2 = jnp.concatenate(res_parts, axis=0) if nhalves > 1 else res_parts[0]
    s3 = jnp.concatenate(s_parts, axis=0) if nhalves > 1 else s_parts[0]
    return res2, s3


# speedup vs baseline: 1.2571x; 1.2571x over previous
import functools

import jax
import jax.numpy as jnp
from jax.experimental import pallas as pl
from jax.experimental.pallas import tpu as pltpu

_INTERPRET = False


def _encoder_kernel(xh_ref, w1h_ref, b1_ref, hh_ref):
    acc = jnp.dot(xh_ref[...], w1h_ref[...],
                  preferred_element_type=jnp.float32)
    hh_ref[...] = jnp.tanh(acc + b1_ref[...]).astype(jnp.bfloat16)


def _main_kernel(hh_ref, w2h_ref, b2_ref, g_ref, cb_ref, s_ref, res_ref,
                 zbuf_ref, *, nm, K, nj):
    j = pl.program_id(1)
    par = jax.lax.rem(j, 2)

    zprev = zbuf_ref[1 - par]
    a = jnp.maximum(zprev + b2_ref[...], 0.0)
    zg = a + g_ref[...]
    for mi in range(nm):
        zm = zg[:, mi * K:(mi + 1) * K]
        mx = jnp.max(zm, axis=-1, keepdims=True)
        iota = jax.lax.broadcasted_iota(jnp.int32, zm.shape, 1).astype(
            jnp.float32)
        idx = jnp.min(jnp.where(zm == mx, iota, float(K)), axis=-1,
                      keepdims=True)
        oh = (iota == idx).astype(jnp.float32)
        s_ref[:, mi * K:(mi + 1) * K] = oh

    part = jnp.dot(hh_ref[...], w2h_ref[...],
                   preferred_element_type=jnp.float32)
    zbuf_ref[par] = part

    for mi in range(nm):
        ohb = s_ref[:, mi * K:(mi + 1) * K].astype(jnp.bfloat16)
        r = jnp.dot(ohb, cb_ref[mi], preferred_element_type=jnp.float32)
        sub = r.shape[-1] // 2
        res_ref[0, :, mi * sub:(mi + 1) * sub] = r[:, :sub] + r[:, sub:]


def kernel(x, W1, b1, W2, b2, codebook, gumbel):
    B, DIM = x.shape
    MK2 = W1.shape[1]
    MK = W2.shape[1]
    M, K, SUB = codebook.shape

    xh = x.astype(jnp.bfloat16)
    w1h = W1.astype(jnp.bfloat16)
    w2h = W2.astype(jnp.bfloat16)
    cbh = codebook.astype(jnp.bfloat16)
    cbl = (codebook - cbh.astype(jnp.float32)).astype(jnp.bfloat16)
    cb_cat = jnp.concatenate([cbh, cbl], axis=-1)

    B1 = 1024 if B % 1024 == 0 else B
    grid_a = (B // B1,)
    hh = pl.pallas_call(
        _encoder_kernel,
        grid=grid_a,
        in_specs=[
            pl.BlockSpec((B1, DIM), lambda i: (i, 0)),
            pl.BlockSpec((DIM, MK2), lambda i: (0, 0)),
            pl.BlockSpec((1, MK2), lambda i: (0, 0)),
        ],
        out_specs=pl.BlockSpec((B1, MK2), lambda i: (i, 0)),
        out_shape=jax.ShapeDtypeStruct((B, MK2), jnp.bfloat16),
        compiler_params=pltpu.CompilerParams(
            dimension_semantics=("parallel",),
            vmem_limit_bytes=50 * 1024 * 1024,
        ),
        name="wolfpq_encoder",
        interpret=_INTERPRET,
    )(xh, w1h, b1.reshape(1, MK2))

    BB = 1024 if B % 1024 == 0 else B
    NB = 512 if MK % 512 == 0 else MK
    nm = NB // K
    nj = MK // NB
    grid_b = (B // BB, nj + 1)

    def prev_i(i, j):
        return jnp.where(j == 0, jnp.maximum(i - 1, 0), i)

    def prev_j(j):
        return jax.lax.rem(j + nj - 1, nj)

    def cur_j(j):
        return jnp.minimum(j, nj - 1)

    body = functools.partial(_main_kernel, nm=nm, K=K, nj=nj)
    s2d, res2 = pl.pallas_call(
        body,
        grid=grid_b,
        in_specs=[
            pl.BlockSpec((BB, MK2), lambda i, j: (i, 0)),
            pl.BlockSpec((MK2, NB), lambda i, j: (0, cur_j(j))),
            pl.BlockSpec((1, NB), lambda i, j: (0, prev_j(j))),
            pl.BlockSpec((BB, NB), lambda i, j: (prev_i(i, j), prev_j(j))),
            pl.BlockSpec((nm, K, 2 * SUB), lambda i, j: (prev_j(j), 0, 0)),
        ],
        out_specs=[
            pl.BlockSpec((BB, NB), lambda i, j: (prev_i(i, j), prev_j(j))),
            pl.BlockSpec((1, BB, nm * SUB),
                         lambda i, j: (prev_j(j), prev_i(i, j), 0)),
        ],
        out_shape=[
            jax.ShapeDtypeStruct((B, MK), jnp.float32),
            jax.ShapeDtypeStruct((MK // NB, B, nm * SUB), jnp.float32),
        ],
        scratch_shapes=[pltpu.VMEM((2, BB, NB), jnp.float32)],
        compiler_params=pltpu.CompilerParams(
            dimension_semantics=("parallel", "arbitrary"),
            vmem_limit_bytes=56 * 1024 * 1024,
        ),
        name="wolfpq_vq",
        interpret=_INTERPRET,
    )(hh, w2h, b2.reshape(1, MK), gumbel.reshape(B, MK), cb_cat)

    res2 = res2.transpose(1, 0, 2).reshape(B, M * SUB)
    return res2, s2d.reshape(B, M, K)
